# baseline (speedup 1.0000x reference)
import functools
import math

import jax
import jax.numpy as jnp
from jax.experimental import pallas as pl

B, L, D = 1, 2048, 768
H, HD = 12, 64
LAT = 16
HID = 3072
E, K, S = 8, 2, 2
NE = E + S

TL = 256
HC = 768
NHC = HID // HC


def _ln(x, g, b):
    m = jnp.mean(x, axis=-1, keepdims=True)
    v = jnp.mean((x - m) * (x - m), axis=-1, keepdims=True)
    return (x - m) * jax.lax.rsqrt(v + 1e-5) * g + b


def _dot(a, b):
    return jnp.dot(a, b, preferred_element_type=jnp.float32)


def _k1_body(x_ref, f_ref, g_ref, b_ref, wq_ref, bq_ref, wk_ref, bk_ref,
             wqc_ref, bqc_ref, wkc_ref, bkc_ref, wvl_ref, bvl_ref,
             qc_ref, kc_ref, vc_ref):
    h = _ln(x_ref[:], g_ref[:], b_ref[:]).astype(jnp.bfloat16)
    f = f_ref[:]
    q = ((_dot(h, wq_ref[:]) + bq_ref[:]) * f).astype(jnp.bfloat16)
    k = ((_dot(h, wk_ref[:]) + bk_ref[:]) * f).astype(jnp.bfloat16)
    qc_ref[:] = (_dot(q, wqc_ref[:]) + bqc_ref[:]).astype(jnp.bfloat16)
    kc_ref[:] = (_dot(k, wkc_ref[:]) + bkc_ref[:]).astype(jnp.bfloat16)
    vc_ref[:] = (_dot(h, wvl_ref[:]) + bvl_ref[:]).astype(jnp.bfloat16)


def _k2_body(q_ref, k_ref, v_ref, o_ref):
    i = pl.program_id(1)
    q = q_ref[0]
    kc = k_ref[0]
    s = jax.lax.dot_general(q, kc, (((1,), (1,)), ((), ())),
                            preferred_element_type=jnp.float32)
    s = s * (1.0 / math.sqrt(LAT))
    rows = i * TL + jax.lax.broadcasted_iota(jnp.int32, (TL, L), 0)
    cols = jax.lax.broadcasted_iota(jnp.int32, (TL, L), 1)
    s = jnp.where(cols > rows, -1e30, s)
    m = jnp.max(s, axis=1, keepdims=True)
    p = jnp.exp(s - m)
    denom = jnp.sum(p, axis=1, keepdims=True)
    o_ref[0] = (_dot(p.astype(jnp.bfloat16), v_ref[0]) /
                denom).astype(jnp.bfloat16)


def _k3_body(ao_ref, x_ref, wdo_ref, batt_ref, g_ref, b_ref, wr_ref, br_ref,
             x1_ref, h2_ref, cmb_ref):
    x1 = x_ref[:] + _dot(ao_ref[:], wdo_ref[:]) + batt_ref[:]
    x1_ref[:] = x1
    h2 = _ln(x1, g_ref[:], b_ref[:])
    h2_ref[:] = h2.astype(jnp.bfloat16)
    gts = _dot(h2, wr_ref[:]) + br_ref[:]
    lane = jax.lax.broadcasted_iota(jnp.int32, (TL, E), 1)
    m1 = jnp.max(gts, axis=1, keepdims=True)
    i1 = jnp.argmax(gts, axis=1).reshape(TL, 1)
    g2 = jnp.where(lane == i1, -1e30, gts)
    m2 = jnp.max(g2, axis=1, keepdims=True)
    i2 = jnp.argmax(g2, axis=1).reshape(TL, 1)
    bb = jnp.exp(m2 - m1)
    w1 = 1.0 / (1.0 + bb)
    w2 = 1.0 - w1
    cmb8 = w1 * (lane == i1).astype(jnp.float32) + \
        w2 * (lane == i2).astype(jnp.float32)
    shared = jnp.full((TL, S), 1.0 / S, jnp.float32)
    pad = jnp.zeros((TL, 16 - E - S), jnp.float32)
    cmb_ref[:] = jnp.concatenate([cmb8, shared, pad], axis=1)


def _k4_body(h2_ref, x1_ref, cmb_ref, we1_ref, ws1_ref, b1_ref,
             we2_ref, ws2_ref, b2_ref, o_ref):
    e = pl.program_id(0)
    c = pl.program_id(1)

    @pl.when((e == 0) & (c == 0))
    def _():
        o_ref[:] = x1_ref[:]

    w1 = jax.lax.cond(e < E, lambda: we1_ref[0], lambda: ws1_ref[0])
    w2 = jax.lax.cond(e < E, lambda: we2_ref[0], lambda: ws2_ref[0])
    h = _dot(h2_ref[:], w1) + b1_ref[0]
    h = 0.5 * h * (1.0 + jax.lax.erf(h * (1.0 / math.sqrt(2.0))))
    contrib = _dot(h.astype(jnp.bfloat16), w2)
    contrib = contrib + jnp.where(c == 0, 1.0, 0.0) * b2_ref[0]
    lane = jax.lax.broadcasted_iota(jnp.int32, (L, 16), 1)
    w = jnp.sum(cmb_ref[:] * (lane == e).astype(jnp.float32),
                axis=1, keepdims=True)
    o_ref[:] += w * contrib


def kernel(x, freqs_cis, ln1_g, ln1_b, Wq, bq, Wk, bk, Wv, bv, Wqc, bqc,
           Wkc, bkc, Wvc, bvc, Wd, bd, Wo, bo, ln2_g, ln2_b, Wr, br,
           We1, be1, We2, be2, Ws1, bs1, Ws2, bs2):
    xf = x.reshape(L, D)

    bf = jnp.bfloat16
    F = jnp.repeat(freqs_cis, 2, axis=1)
    F_full = jnp.tile(F, (1, H))
    Wq_b = Wq.astype(bf)
    Wk_b = Wk.astype(bf)
    Wqc_bd = jnp.kron(jnp.eye(H, dtype=jnp.float32), Wqc).astype(bf)
    Wkc_bd = jnp.kron(jnp.eye(H, dtype=jnp.float32), Wkc).astype(bf)
    bqc_t = jnp.tile(bqc, H).reshape(1, H * LAT)
    bkc_t = jnp.tile(bkc, H).reshape(1, H * LAT)
    Wv_lat = jnp.einsum('dhk,kl->dhl', Wv.reshape(D, H, HD),
                        Wvc).reshape(D, H * LAT).astype(bf)
    bv_lat = (bv.reshape(H, HD) @ Wvc + bvc[None]).reshape(1, H * LAT)
    Wdo = jnp.einsum('lk,hkd->hld', Wd,
                     Wo.reshape(H, HD, D)).reshape(H * LAT, D).astype(bf)
    b_att = (jnp.tile(bd, H) @ Wo + bo).reshape(1, D)
    b1_all = jnp.concatenate([be1, bs1], axis=0).reshape(NE, 1, HID)
    b2_all = jnp.concatenate([be2, bs2], axis=0).reshape(NE, 1, D)

    r1 = lambda a: a.reshape(1, -1)
    NT = L // TL

    qc, kc, vc = pl.pallas_call(
        _k1_body,
        grid=(NT,),
        in_specs=[
            pl.BlockSpec((TL, D), lambda i: (i, 0)),
            pl.BlockSpec((TL, D), lambda i: (i, 0)),
            pl.BlockSpec((1, D), lambda i: (0, 0)),
            pl.BlockSpec((1, D), lambda i: (0, 0)),
            pl.BlockSpec((D, D), lambda i: (0, 0)),
            pl.BlockSpec((1, D), lambda i: (0, 0)),
            pl.BlockSpec((D, D), lambda i: (0, 0)),
            pl.BlockSpec((1, D), lambda i: (0, 0)),
            pl.BlockSpec((D, H * LAT), lambda i: (0, 0)),
            pl.BlockSpec((1, H * LAT), lambda i: (0, 0)),
            pl.BlockSpec((D, H * LAT), lambda i: (0, 0)),
            pl.BlockSpec((1, H * LAT), lambda i: (0, 0)),
            pl.BlockSpec((D, H * LAT), lambda i: (0, 0)),
            pl.BlockSpec((1, H * LAT), lambda i: (0, 0)),
        ],
        out_specs=[
            pl.BlockSpec((TL, H * LAT), lambda i: (i, 0)),
            pl.BlockSpec((TL, H * LAT), lambda i: (i, 0)),
            pl.BlockSpec((TL, H * LAT), lambda i: (i, 0)),
        ],
        out_shape=[jax.ShapeDtypeStruct((L, H * LAT), bf)] * 3,
    )(xf, F_full, r1(ln1_g), r1(ln1_b), Wq_b, r1(bq), Wk_b, r1(bk),
      Wqc_bd, bqc_t, Wkc_bd, bkc_t, Wv_lat, bv_lat)

    qc3 = qc.reshape(L, H, LAT).transpose(1, 0, 2)
    kc3 = kc.reshape(L, H, LAT).transpose(1, 0, 2)
    vc3 = vc.reshape(L, H, LAT).transpose(1, 0, 2)

    ao3 = pl.pallas_call(
        _k2_body,
        grid=(H, NT),
        in_specs=[
            pl.BlockSpec((1, TL, LAT), lambda h, i: (h, i, 0)),
            pl.BlockSpec((1, L, LAT), lambda h, i: (h, 0, 0)),
            pl.BlockSpec((1, L, LAT), lambda h, i: (h, 0, 0)),
        ],
        out_specs=pl.BlockSpec((1, TL, LAT), lambda h, i: (h, i, 0)),
        out_shape=jax.ShapeDtypeStruct((H, L, LAT), bf),
    )(qc3, kc3, vc3)

    ao_flat = ao3.transpose(1, 0, 2).reshape(L, H * LAT)

    x1, h2, cmb = pl.pallas_call(
        _k3_body,
        grid=(NT,),
        in_specs=[
            pl.BlockSpec((TL, H * LAT), lambda i: (i, 0)),
            pl.BlockSpec((TL, D), lambda i: (i, 0)),
            pl.BlockSpec((H * LAT, D), lambda i: (0, 0)),
            pl.BlockSpec((1, D), lambda i: (0, 0)),
            pl.BlockSpec((1, D), lambda i: (0, 0)),
            pl.BlockSpec((1, D), lambda i: (0, 0)),
            pl.BlockSpec((D, E), lambda i: (0, 0)),
            pl.BlockSpec((1, E), lambda i: (0, 0)),
        ],
        out_specs=[
            pl.BlockSpec((TL, D), lambda i: (i, 0)),
            pl.BlockSpec((TL, D), lambda i: (i, 0)),
            pl.BlockSpec((TL, 16), lambda i: (i, 0)),
        ],
        out_shape=[
            jax.ShapeDtypeStruct((L, D), jnp.float32),
            jax.ShapeDtypeStruct((L, D), bf),
            jax.ShapeDtypeStruct((L, 16), jnp.float32),
        ],
    )(ao_flat, xf, Wdo, b_att, r1(ln2_g), r1(ln2_b), Wr, r1(br))

    out = pl.pallas_call(
        _k4_body,
        grid=(NE, NHC),
        in_specs=[
            pl.BlockSpec((L, D), lambda e, c: (0, 0)),
            pl.BlockSpec((L, D), lambda e, c: (0, 0)),
            pl.BlockSpec((L, 16), lambda e, c: (0, 0)),
            pl.BlockSpec((1, D, HC),
                         lambda e, c: (jnp.minimum(e, E - 1), 0, c)),
            pl.BlockSpec((1, D, HC),
                         lambda e, c: (jnp.maximum(e - E, 0), 0, c)),
            pl.BlockSpec((1, 1, HC), lambda e, c: (e, 0, c)),
            pl.BlockSpec((1, HC, D),
                         lambda e, c: (jnp.minimum(e, E - 1), c, 0)),
            pl.BlockSpec((1, HC, D),
                         lambda e, c: (jnp.maximum(e - E, 0), c, 0)),
            pl.BlockSpec((1, 1, D), lambda e, c: (e, 0, 0)),
        ],
        out_specs=pl.BlockSpec((L, D), lambda e, c: (0, 0)),
        out_shape=jax.ShapeDtypeStruct((L, D), jnp.float32),
    )(h2, x1, cmb, We1.astype(bf), Ws1.astype(bf), b1_all,
      We2.astype(bf), Ws2.astype(bf), b2_all)

    return out.reshape(B, L, D)


# speedup vs baseline: 1.1218x; 1.1218x over previous
import functools
import math

import jax
import jax.numpy as jnp
from jax.experimental import pallas as pl

B, L, D = 1, 2048, 768
H, HD = 12, 64
LAT = 16
HID = 3072
E, K, S = 8, 2, 2
NE = E + S

TL = 256
HC = 768
NHC = HID // HC


def _ln(x, g, b):
    m = jnp.mean(x, axis=-1, keepdims=True)
    v = jnp.mean((x - m) * (x - m), axis=-1, keepdims=True)
    return (x - m) * jax.lax.rsqrt(v + 1e-5) * g + b


def _dot(a, b):
    return jnp.dot(a, b, preferred_element_type=jnp.float32)


def _k1_body(x_ref, f_ref, g_ref, b_ref, wq_ref, bq_ref, wk_ref, bk_ref,
             wqc_ref, bqc_ref, wkc_ref, bkc_ref, wvl_ref, bvl_ref,
             qc_ref, kc_ref, vc_ref):
    h = _ln(x_ref[:], g_ref[:], b_ref[:]).astype(jnp.bfloat16)
    f = f_ref[:]
    q = ((_dot(h, wq_ref[:]) + bq_ref[:]) * f).astype(jnp.bfloat16)
    k = ((_dot(h, wk_ref[:]) + bk_ref[:]) * f).astype(jnp.bfloat16)
    qc_ref[:] = (_dot(q, wqc_ref[:]) + bqc_ref[:]).astype(jnp.bfloat16)
    kc_ref[:] = (_dot(k, wkc_ref[:]) + bkc_ref[:]).astype(jnp.bfloat16)
    vc_ref[:] = (_dot(h, wvl_ref[:]) + bvl_ref[:]).astype(jnp.bfloat16)


def _k2_body(q_ref, k_ref, v_ref, o_ref):
    i = pl.program_id(1)
    q = q_ref[0]
    kc = k_ref[0]
    s = jax.lax.dot_general(q, kc, (((1,), (1,)), ((), ())),
                            preferred_element_type=jnp.float32)
    s = s * (1.0 / math.sqrt(LAT))
    rows = i * TL + jax.lax.broadcasted_iota(jnp.int32, (TL, L), 0)
    cols = jax.lax.broadcasted_iota(jnp.int32, (TL, L), 1)
    s = jnp.where(cols > rows, -1e30, s)
    m = jnp.max(s, axis=1, keepdims=True)
    p = jnp.exp(s - m)
    denom = jnp.sum(p, axis=1, keepdims=True)
    o_ref[0] = (_dot(p.astype(jnp.bfloat16), v_ref[0]) /
                denom).astype(jnp.bfloat16)


def _k3_body(ao_ref, x_ref, wdo_ref, batt_ref, g_ref, b_ref, wr_ref, br_ref,
             x1_ref, h2_ref, cmb_ref):
    x1 = x_ref[:] + _dot(ao_ref[:], wdo_ref[:]) + batt_ref[:]
    x1_ref[:] = x1
    h2 = _ln(x1, g_ref[:], b_ref[:])
    h2_ref[:] = h2.astype(jnp.bfloat16)
    gts = _dot(h2, wr_ref[:]) + br_ref[:]
    lane = jax.lax.broadcasted_iota(jnp.int32, (TL, E), 1)
    m1 = jnp.max(gts, axis=1, keepdims=True)
    i1 = jnp.argmax(gts, axis=1).reshape(TL, 1)
    g2 = jnp.where(lane == i1, -1e30, gts)
    m2 = jnp.max(g2, axis=1, keepdims=True)
    i2 = jnp.argmax(g2, axis=1).reshape(TL, 1)
    bb = jnp.exp(m2 - m1)
    w1 = 1.0 / (1.0 + bb)
    w2 = 1.0 - w1
    cmb8 = w1 * (lane == i1).astype(jnp.float32) + \
        w2 * (lane == i2).astype(jnp.float32)
    shared = jnp.full((TL, S), 1.0 / S, jnp.float32)
    pad = jnp.zeros((TL, 16 - E - S), jnp.float32)
    cmb_ref[:] = jnp.concatenate([cmb8, shared, pad], axis=1)


def _k4_body(h2_ref, x1_ref, cmb_ref, we1_ref, ws1_ref, b1_ref,
             we2_ref, ws2_ref, b2_ref, o_ref):
    e = pl.program_id(0)
    c = pl.program_id(1)

    @pl.when((e == 0) & (c == 0))
    def _():
        o_ref[:] = x1_ref[:]

    w1 = jax.lax.cond(e < E, lambda: we1_ref[0],
                      lambda: ws1_ref[0]).astype(jnp.bfloat16)
    w2 = jax.lax.cond(e < E, lambda: we2_ref[0],
                      lambda: ws2_ref[0]).astype(jnp.bfloat16)
    h = _dot(h2_ref[:], w1) + b1_ref[0]
    h = 0.5 * h * (1.0 + jax.lax.erf(h * (1.0 / math.sqrt(2.0))))
    contrib = _dot(h.astype(jnp.bfloat16), w2)
    contrib = contrib + jnp.where(c == 0, 1.0, 0.0) * b2_ref[0]
    lane = jax.lax.broadcasted_iota(jnp.int32, (L, 16), 1)
    w = jnp.sum(cmb_ref[:] * (lane == e).astype(jnp.float32),
                axis=1, keepdims=True)
    o_ref[:] += w * contrib


def kernel(x, freqs_cis, ln1_g, ln1_b, Wq, bq, Wk, bk, Wv, bv, Wqc, bqc,
           Wkc, bkc, Wvc, bvc, Wd, bd, Wo, bo, ln2_g, ln2_b, Wr, br,
           We1, be1, We2, be2, Ws1, bs1, Ws2, bs2):
    xf = x.reshape(L, D)

    bf = jnp.bfloat16
    F = jnp.repeat(freqs_cis, 2, axis=1)
    F_full = jnp.tile(F, (1, H))
    Wq_b = Wq.astype(bf)
    Wk_b = Wk.astype(bf)
    Wqc_bd = jnp.kron(jnp.eye(H, dtype=jnp.float32), Wqc).astype(bf)
    Wkc_bd = jnp.kron(jnp.eye(H, dtype=jnp.float32), Wkc).astype(bf)
    bqc_t = jnp.tile(bqc, H).reshape(1, H * LAT)
    bkc_t = jnp.tile(bkc, H).reshape(1, H * LAT)
    Wv_lat = jnp.einsum('dhk,kl->dhl', Wv.reshape(D, H, HD),
                        Wvc).reshape(D, H * LAT).astype(bf)
    bv_lat = (bv.reshape(H, HD) @ Wvc + bvc[None]).reshape(1, H * LAT)
    Wdo = jnp.einsum('lk,hkd->hld', Wd,
                     Wo.reshape(H, HD, D)).reshape(H * LAT, D).astype(bf)
    b_att = (jnp.tile(bd, H) @ Wo + bo).reshape(1, D)
    b1_all = jnp.concatenate([be1, bs1], axis=0).reshape(NE, 1, HID)
    b2_all = jnp.concatenate([be2, bs2], axis=0).reshape(NE, 1, D)

    r1 = lambda a: a.reshape(1, -1)
    NT = L // TL

    qc, kc, vc = pl.pallas_call(
        _k1_body,
        grid=(NT,),
        in_specs=[
            pl.BlockSpec((TL, D), lambda i: (i, 0)),
            pl.BlockSpec((TL, D), lambda i: (i, 0)),
            pl.BlockSpec((1, D), lambda i: (0, 0)),
            pl.BlockSpec((1, D), lambda i: (0, 0)),
            pl.BlockSpec((D, D), lambda i: (0, 0)),
            pl.BlockSpec((1, D), lambda i: (0, 0)),
            pl.BlockSpec((D, D), lambda i: (0, 0)),
            pl.BlockSpec((1, D), lambda i: (0, 0)),
            pl.BlockSpec((D, H * LAT), lambda i: (0, 0)),
            pl.BlockSpec((1, H * LAT), lambda i: (0, 0)),
            pl.BlockSpec((D, H * LAT), lambda i: (0, 0)),
            pl.BlockSpec((1, H * LAT), lambda i: (0, 0)),
            pl.BlockSpec((D, H * LAT), lambda i: (0, 0)),
            pl.BlockSpec((1, H * LAT), lambda i: (0, 0)),
        ],
        out_specs=[
            pl.BlockSpec((TL, H * LAT), lambda i: (i, 0)),
            pl.BlockSpec((TL, H * LAT), lambda i: (i, 0)),
            pl.BlockSpec((TL, H * LAT), lambda i: (i, 0)),
        ],
        out_shape=[jax.ShapeDtypeStruct((L, H * LAT), bf)] * 3,
    )(xf, F_full, r1(ln1_g), r1(ln1_b), Wq_b, r1(bq), Wk_b, r1(bk),
      Wqc_bd, bqc_t, Wkc_bd, bkc_t, Wv_lat, bv_lat)

    qc3 = qc.reshape(L, H, LAT).transpose(1, 0, 2)
    kc3 = kc.reshape(L, H, LAT).transpose(1, 0, 2)
    vc3 = vc.reshape(L, H, LAT).transpose(1, 0, 2)

    ao3 = pl.pallas_call(
        _k2_body,
        grid=(H, NT),
        in_specs=[
            pl.BlockSpec((1, TL, LAT), lambda h, i: (h, i, 0)),
            pl.BlockSpec((1, L, LAT), lambda h, i: (h, 0, 0)),
            pl.BlockSpec((1, L, LAT), lambda h, i: (h, 0, 0)),
        ],
        out_specs=pl.BlockSpec((1, TL, LAT), lambda h, i: (h, i, 0)),
        out_shape=jax.ShapeDtypeStruct((H, L, LAT), bf),
    )(qc3, kc3, vc3)

    ao_flat = ao3.transpose(1, 0, 2).reshape(L, H * LAT)

    x1, h2, cmb = pl.pallas_call(
        _k3_body,
        grid=(NT,),
        in_specs=[
            pl.BlockSpec((TL, H * LAT), lambda i: (i, 0)),
            pl.BlockSpec((TL, D), lambda i: (i, 0)),
            pl.BlockSpec((H * LAT, D), lambda i: (0, 0)),
            pl.BlockSpec((1, D), lambda i: (0, 0)),
            pl.BlockSpec((1, D), lambda i: (0, 0)),
            pl.BlockSpec((1, D), lambda i: (0, 0)),
            pl.BlockSpec((D, E), lambda i: (0, 0)),
            pl.BlockSpec((1, E), lambda i: (0, 0)),
        ],
        out_specs=[
            pl.BlockSpec((TL, D), lambda i: (i, 0)),
            pl.BlockSpec((TL, D), lambda i: (i, 0)),
            pl.BlockSpec((TL, 16), lambda i: (i, 0)),
        ],
        out_shape=[
            jax.ShapeDtypeStruct((L, D), jnp.float32),
            jax.ShapeDtypeStruct((L, D), bf),
            jax.ShapeDtypeStruct((L, 16), jnp.float32),
        ],
    )(ao_flat, xf, Wdo, b_att, r1(ln2_g), r1(ln2_b), Wr, r1(br))

    out = pl.pallas_call(
        _k4_body,
        grid=(NE, NHC),
        in_specs=[
            pl.BlockSpec((L, D), lambda e, c: (0, 0)),
            pl.BlockSpec((L, D), lambda e, c: (0, 0)),
            pl.BlockSpec((L, 16), lambda e, c: (0, 0)),
            pl.BlockSpec((1, D, HC),
                         lambda e, c: (jnp.minimum(e, E - 1), 0, c)),
            pl.BlockSpec((1, D, HC),
                         lambda e, c: (jnp.maximum(e - E, 0), 0, c)),
            pl.BlockSpec((1, 1, HC), lambda e, c: (e, 0, c)),
            pl.BlockSpec((1, HC, D),
                         lambda e, c: (jnp.minimum(e, E - 1), c, 0)),
            pl.BlockSpec((1, HC, D),
                         lambda e, c: (jnp.maximum(e - E, 0), c, 0)),
            pl.BlockSpec((1, 1, D), lambda e, c: (e, 0, 0)),
        ],
        out_specs=pl.BlockSpec((L, D), lambda e, c: (0, 0)),
        out_shape=jax.ShapeDtypeStruct((L, D), jnp.float32),
    )(h2, x1, cmb, We1, Ws1, b1_all, We2, Ws2, b2_all)

    return out.reshape(B, L, D)
